# baseline (speedup 1.0000x reference)
import functools
import jax
import jax.numpy as jnp
from jax import lax
from jax.experimental import pallas as pl
from jax.experimental.pallas import tpu as pltpu
from jax.experimental.pallas import tpu_sc as plsc

CHUNK = 128
LANES = 16
NUM_SC = 2
NUM_SUBCORES = 16
SUPER = 32


def _tc_matmul(x, W):
    def body(x_ref, w_ref, o_ref):
        o_ref[...] = jnp.dot(x_ref[...], w_ref[...],
                             preferred_element_type=jnp.float32)

    n, _ = x.shape
    d_out = W.shape[1]
    return pl.pallas_call(
        body,
        out_shape=jax.ShapeDtypeStruct((n, d_out), jnp.float32),
    )(x, W)


def _tc_combine(partials, XW, self_scale, bias, gamma, beta):
    n, d = XW.shape

    def body(p_ref, xw_ref, s_ref, b_ref, g_ref, be_ref, o_ref):
        s = s_ref[0, 0]
        tmp = p_ref[0] + p_ref[1] + xw_ref[...] * s + b_ref[...]
        mean = jnp.mean(tmp, axis=0, keepdims=True)
        var = jnp.mean((tmp - mean) * (tmp - mean), axis=0, keepdims=True)
        inv = lax.rsqrt(var + 1e-5)
        o_ref[...] = (tmp - mean) * inv * g_ref[...] + be_ref[...]

    return pl.pallas_call(
        body,
        in_specs=[
            pl.BlockSpec(memory_space=pltpu.VMEM),
            pl.BlockSpec(memory_space=pltpu.VMEM),
            pl.BlockSpec(memory_space=pltpu.SMEM),
            pl.BlockSpec(memory_space=pltpu.VMEM),
            pl.BlockSpec(memory_space=pltpu.VMEM),
            pl.BlockSpec(memory_space=pltpu.VMEM),
        ],
        out_shape=jax.ShapeDtypeStruct((n, d), jnp.float32),
    )(partials, XW, self_scale, bias, gamma, beta)


def _make_sc_aggregate(n_nodes, d, n_chunks_per_worker, alpha_len):
    nc, ns = NUM_SC, NUM_SUBCORES
    rows_per_tile = (n_nodes // ns) // 8 * 8
    rows_rem = n_nodes - rows_per_tile * ns
    mesh = plsc.VectorSubcoreMesh(core_axis_name="c", subcore_axis_name="s",
                                  num_cores=nc, num_subcores=ns)

    n_chunks = n_chunks_per_worker
    n_sup = n_chunks // SUPER
    assert n_chunks == n_sup * SUPER

    @functools.partial(
        pl.kernel,
        mesh=mesh,
        compiler_params=pltpu.CompilerParams(needs_layout_passes=False),
        out_type=jax.ShapeDtypeStruct((nc, n_nodes, d), jnp.float32),
        scratch_types=[
            pltpu.VMEM((SUPER, CHUNK), jnp.int32),
            pltpu.VMEM((SUPER, CHUNK), jnp.int32),
            pltpu.VMEM((SUPER, CHUNK), jnp.int32),
            pltpu.VMEM((CHUNK, d), jnp.float32),
            pltpu.VMEM((CHUNK, d), jnp.float32),
            pltpu.VMEM((alpha_len,), jnp.float32),
            pltpu.VMEM_SHARED((n_nodes, d), jnp.float32),
            pltpu.SemaphoreType.DMA,
            pltpu.SemaphoreType.DMA,
        ],
    )
    def sc_agg(xw_hbm, src_hbm, dst_hbm, et_hbm, alpha_hbm, out_hbm,
               src_v, dst_v, et_v, rows0, rows1, alpha_v, acc_sh, sem,
               sem_s):
        cid = lax.axis_index("c")
        sid = lax.axis_index("s")
        wid = sid * nc + cid

        def zero_rows(i, _):
            for k in range(d // LANES):
                rows0[i, pl.ds(k * LANES, LANES)] = jnp.zeros(
                    (LANES,), jnp.float32)
            return 0
        lax.fori_loop(0, CHUNK, zero_rows, 0)

        my_base = sid * rows_per_tile
        full = rows_per_tile // CHUNK
        for j in range(full):
            pltpu.sync_copy(rows0,
                            acc_sh.at[pl.ds(my_base + j * CHUNK, CHUNK)])
        tail = rows_per_tile - full * CHUNK
        if tail:
            pltpu.sync_copy(rows0.at[pl.ds(0, tail)],
                            acc_sh.at[pl.ds(my_base + full * CHUNK, tail)])
        if rows_rem:
            @pl.when(sid == 0)
            def _():
                pltpu.sync_copy(
                    rows0.at[pl.ds(0, rows_rem)],
                    acc_sh.at[pl.ds(ns * rows_per_tile, rows_rem)])

        pltpu.sync_copy(alpha_hbm, alpha_v)
        plsc.subcore_barrier()

        def scale_and_scatter(g, buf):
            def scale_grp(grp, _):
                idx = et_v[g, pl.ds(grp * LANES, LANES)]
                a16 = plsc.load_gather(alpha_v, [idx])
                for j in range(LANES):
                    a_s = a16[j]
                    for k in range(d // LANES):
                        sl = pl.ds(k * LANES, LANES)
                        buf[grp * LANES + j, sl] = \
                            buf[grp * LANES + j, sl] * a_s
                return 0
            lax.fori_loop(0, CHUNK // LANES, scale_grp, 0)
            pltpu.async_copy(buf, acc_sh.at[dst_v.at[g]], sem_s, add=True)

        def outer_s(s, _):
            base_c = wid * n_chunks + s * SUPER
            pltpu.sync_copy(src_hbm.at[pl.ds(base_c, SUPER)], src_v)
            pltpu.sync_copy(dst_hbm.at[pl.ds(base_c, SUPER)], dst_v)
            pltpu.sync_copy(et_hbm.at[pl.ds(base_c, SUPER)], et_v)
            pltpu.async_copy(xw_hbm.at[src_v.at[0]], rows0, sem)

            def inner(g2, _):
                g0 = g2 * 2
                for b in range(2):
                    g = g0 + b
                    cur = rows0 if b == 0 else rows1
                    nxt = rows1 if b == 0 else rows0

                    pltpu.make_async_copy(xw_hbm.at[src_v.at[g]], cur,
                                          sem).wait()
                    @pl.when(g > 0)
                    def _():
                        pltpu.make_async_copy(
                            nxt, acc_sh.at[dst_v.at[g]], sem_s).wait()

                    @pl.when(g + 1 < SUPER)
                    def _():
                        pltpu.async_copy(xw_hbm.at[src_v.at[g + 1]], nxt,
                                         sem)
                    scale_and_scatter(g, cur)
                return 0
            lax.fori_loop(0, SUPER // 2, inner, 0)
            pltpu.make_async_copy(rows1, acc_sh.at[dst_v.at[SUPER - 1]],
                                  sem_s).wait()
            return 0
        lax.fori_loop(0, n_sup, outer_s, 0)

        plsc.subcore_barrier()
        pltpu.sync_copy(acc_sh.at[pl.ds(my_base, rows_per_tile)],
                        out_hbm.at[cid, pl.ds(my_base, rows_per_tile)])
        if rows_rem:
            @pl.when(sid == 0)
            def _():
                pltpu.sync_copy(
                    acc_sh.at[pl.ds(ns * rows_per_tile, rows_rem)],
                    out_hbm.at[cid, pl.ds(ns * rows_per_tile, rows_rem)])

    return sc_agg


def kernel(x, r, edge_index, edge_type, W, alpha, bias, gamma, beta):
    n, d_in = x.shape
    d_out = W.shape[1]
    e = edge_index.shape[1]
    self_rel = r.shape[0] - 1

    n_workers = NUM_SC * NUM_SUBCORES

    XW = _tc_matmul(x, W)

    row, col = edge_index[0], edge_index[1]
    src = jnp.concatenate([col, row])
    dst = jnp.concatenate([row, col])
    et2 = jnp.concatenate([edge_type, edge_type])

    pad_rel = alpha.shape[0]
    total = 2 * e
    chunks_per_worker = -(-total // (n_workers * CHUNK))
    chunks_per_worker = -(-chunks_per_worker // SUPER) * SUPER
    per_worker = chunks_per_worker * CHUNK
    padded = per_worker * n_workers
    pad = padded - total
    spread = jnp.arange(pad, dtype=jnp.int32) % n
    src = jnp.concatenate([src, spread]).reshape(-1, CHUNK)
    dst = jnp.concatenate([dst, spread]).reshape(-1, CHUNK)
    et2 = jnp.pad(et2, (0, pad), constant_values=pad_rel).reshape(-1, CHUNK)

    alpha_len = -(-(pad_rel + 1) // LANES) * LANES
    alpha_vec = jnp.pad(alpha[:, 0], (0, alpha_len - pad_rel))

    sc_agg = _make_sc_aggregate(n, d_out, per_worker // CHUNK, alpha_len)
    partials = sc_agg(XW, src, dst, et2, alpha_vec)

    self_scale = (2.0 * alpha[self_rel, 0]).reshape(1, 1)
    out = _tc_combine(partials, XW, self_scale, bias, gamma, beta)
    return (out, r)


# speedup vs baseline: 1.0032x; 1.0032x over previous
import functools
import jax
import jax.numpy as jnp
from jax import lax
from jax.experimental import pallas as pl
from jax.experimental.pallas import tpu as pltpu
from jax.experimental.pallas import tpu_sc as plsc

CHUNK = 64
LANES = 16
NUM_SC = 2
NUM_SUBCORES = 16
SUPER = 16


def _tc_matmul(x, W):
    def body(x_ref, w_ref, o_ref):
        o_ref[...] = jnp.dot(x_ref[...], w_ref[...],
                             preferred_element_type=jnp.float32)

    n, _ = x.shape
    d_out = W.shape[1]
    return pl.pallas_call(
        body,
        out_shape=jax.ShapeDtypeStruct((n, d_out), jnp.float32),
    )(x, W)


def _tc_combine(partials, XW, self_scale, bias, gamma, beta):
    n, d = XW.shape

    def body(p_ref, xw_ref, s_ref, b_ref, g_ref, be_ref, o_ref):
        s = s_ref[0, 0]
        tmp = p_ref[0] + p_ref[1] + xw_ref[...] * s + b_ref[...]
        mean = jnp.mean(tmp, axis=0, keepdims=True)
        var = jnp.mean((tmp - mean) * (tmp - mean), axis=0, keepdims=True)
        inv = lax.rsqrt(var + 1e-5)
        o_ref[...] = (tmp - mean) * inv * g_ref[...] + be_ref[...]

    return pl.pallas_call(
        body,
        in_specs=[
            pl.BlockSpec(memory_space=pltpu.VMEM),
            pl.BlockSpec(memory_space=pltpu.VMEM),
            pl.BlockSpec(memory_space=pltpu.SMEM),
            pl.BlockSpec(memory_space=pltpu.VMEM),
            pl.BlockSpec(memory_space=pltpu.VMEM),
            pl.BlockSpec(memory_space=pltpu.VMEM),
        ],
        out_shape=jax.ShapeDtypeStruct((n, d), jnp.float32),
    )(partials, XW, self_scale, bias, gamma, beta)


def _make_sc_aggregate(n_nodes, d, n_chunks_per_worker, alpha_len):
    nc, ns = NUM_SC, NUM_SUBCORES
    rows_per_tile = (n_nodes // ns) // 8 * 8
    rows_rem = n_nodes - rows_per_tile * ns
    mesh = plsc.VectorSubcoreMesh(core_axis_name="c", subcore_axis_name="s",
                                  num_cores=nc, num_subcores=ns)

    n_chunks = n_chunks_per_worker
    n_sup = n_chunks // SUPER
    assert n_chunks == n_sup * SUPER

    @functools.partial(
        pl.kernel,
        mesh=mesh,
        compiler_params=pltpu.CompilerParams(needs_layout_passes=False),
        out_type=jax.ShapeDtypeStruct((nc, n_nodes, d), jnp.float32),
        scratch_types=[
            pltpu.VMEM((SUPER, CHUNK), jnp.int32),
            pltpu.VMEM((SUPER, CHUNK), jnp.int32),
            pltpu.VMEM((SUPER, CHUNK), jnp.int32),
            pltpu.VMEM((CHUNK, d), jnp.float32),
            pltpu.VMEM((CHUNK, d), jnp.float32),
            pltpu.VMEM((CHUNK, d), jnp.float32),
            pltpu.VMEM((CHUNK, d), jnp.float32),
            pltpu.VMEM((alpha_len,), jnp.float32),
            pltpu.VMEM_SHARED((n_nodes, d), jnp.float32),
            pltpu.SemaphoreType.DMA,
            pltpu.SemaphoreType.DMA,
        ],
    )
    def sc_agg(xw_hbm, src_hbm, dst_hbm, et_hbm, alpha_hbm, out_hbm,
               src_v, dst_v, et_v, rows0, rows1, rows2, rows3, alpha_v,
               acc_sh, sem, sem_s):
        cid = lax.axis_index("c")
        sid = lax.axis_index("s")
        wid = sid * nc + cid

        def zero_rows(i, _):
            for k in range(d // LANES):
                rows0[i, pl.ds(k * LANES, LANES)] = jnp.zeros(
                    (LANES,), jnp.float32)
            return 0
        lax.fori_loop(0, CHUNK, zero_rows, 0)

        my_base = sid * rows_per_tile
        full = rows_per_tile // CHUNK
        for j in range(full):
            pltpu.sync_copy(rows0,
                            acc_sh.at[pl.ds(my_base + j * CHUNK, CHUNK)])
        tail = rows_per_tile - full * CHUNK
        if tail:
            pltpu.sync_copy(rows0.at[pl.ds(0, tail)],
                            acc_sh.at[pl.ds(my_base + full * CHUNK, tail)])
        if rows_rem:
            @pl.when(sid == 0)
            def _():
                pltpu.sync_copy(
                    rows0.at[pl.ds(0, rows_rem)],
                    acc_sh.at[pl.ds(ns * rows_per_tile, rows_rem)])

        pltpu.sync_copy(alpha_hbm, alpha_v)
        plsc.subcore_barrier()

        def scale_and_scatter(g, buf):
            def scale_grp(grp, _):
                idx = et_v[g, pl.ds(grp * LANES, LANES)]
                a16 = plsc.load_gather(alpha_v, [idx])
                for j in range(LANES):
                    a_s = a16[j]
                    for k in range(d // LANES):
                        sl = pl.ds(k * LANES, LANES)
                        buf[grp * LANES + j, sl] = \
                            buf[grp * LANES + j, sl] * a_s
                return 0
            lax.fori_loop(0, CHUNK // LANES, scale_grp, 0)
            pltpu.async_copy(buf, acc_sh.at[dst_v.at[g]], sem_s, add=True)

        bufs = (rows0, rows1, rows2, rows3)

        def outer_s(s, _):
            base_c = wid * n_chunks + s * SUPER
            pltpu.sync_copy(src_hbm.at[pl.ds(base_c, SUPER)], src_v)
            pltpu.sync_copy(dst_hbm.at[pl.ds(base_c, SUPER)], dst_v)
            pltpu.sync_copy(et_hbm.at[pl.ds(base_c, SUPER)], et_v)
            pltpu.async_copy(xw_hbm.at[src_v.at[0]], rows0, sem)
            pltpu.async_copy(xw_hbm.at[src_v.at[1]], rows1, sem)

            def inner(g4, _):
                g0 = g4 * 4
                for b in range(4):
                    g = g0 + b
                    cur = bufs[b]
                    nx2 = bufs[(b + 2) % 4]

                    @pl.when(g >= 2)
                    def _():
                        pltpu.make_async_copy(
                            nx2, acc_sh.at[dst_v.at[g]], sem_s).wait()

                    @pl.when(g + 2 < SUPER)
                    def _():
                        pltpu.async_copy(xw_hbm.at[src_v.at[g + 2]], nx2,
                                         sem)
                    pltpu.make_async_copy(xw_hbm.at[src_v.at[g]], cur,
                                          sem).wait()
                    scale_and_scatter(g, cur)
                return 0
            lax.fori_loop(0, SUPER // 4, inner, 0)
            pltpu.make_async_copy(bufs[(SUPER - 2) % 4],
                                  acc_sh.at[dst_v.at[SUPER - 2]],
                                  sem_s).wait()
            pltpu.make_async_copy(bufs[(SUPER - 1) % 4],
                                  acc_sh.at[dst_v.at[SUPER - 1]],
                                  sem_s).wait()
            return 0
        lax.fori_loop(0, n_sup, outer_s, 0)

        plsc.subcore_barrier()
        pltpu.sync_copy(acc_sh.at[pl.ds(my_base, rows_per_tile)],
                        out_hbm.at[cid, pl.ds(my_base, rows_per_tile)])
        if rows_rem:
            @pl.when(sid == 0)
            def _():
                pltpu.sync_copy(
                    acc_sh.at[pl.ds(ns * rows_per_tile, rows_rem)],
                    out_hbm.at[cid, pl.ds(ns * rows_per_tile, rows_rem)])

    return sc_agg


def kernel(x, r, edge_index, edge_type, W, alpha, bias, gamma, beta):
    n, d_in = x.shape
    d_out = W.shape[1]
    e = edge_index.shape[1]
    self_rel = r.shape[0] - 1

    n_workers = NUM_SC * NUM_SUBCORES

    XW = _tc_matmul(x, W)

    row, col = edge_index[0], edge_index[1]
    src = jnp.concatenate([col, row])
    dst = jnp.concatenate([row, col])
    et2 = jnp.concatenate([edge_type, edge_type])

    pad_rel = alpha.shape[0]
    total = 2 * e
    chunks_per_worker = -(-total // (n_workers * CHUNK))
    chunks_per_worker = -(-chunks_per_worker // SUPER) * SUPER
    per_worker = chunks_per_worker * CHUNK
    padded = per_worker * n_workers
    pad = padded - total
    spread = jnp.arange(pad, dtype=jnp.int32) % n
    src = jnp.concatenate([src, spread]).reshape(-1, CHUNK)
    dst = jnp.concatenate([dst, spread]).reshape(-1, CHUNK)
    et2 = jnp.pad(et2, (0, pad), constant_values=pad_rel).reshape(-1, CHUNK)

    alpha_len = -(-(pad_rel + 1) // LANES) * LANES
    alpha_vec = jnp.pad(alpha[:, 0], (0, alpha_len - pad_rel))

    sc_agg = _make_sc_aggregate(n, d_out, per_worker // CHUNK, alpha_len)
    partials = sc_agg(XW, src, dst, et2, alpha_vec)

    self_scale = (2.0 * alpha[self_rel, 0]).reshape(1, 1)
    out = _tc_combine(partials, XW, self_scale, bias, gamma, beta)
    return (out, r)


# speedup vs baseline: 1.0302x; 1.0270x over previous
import functools
import jax
import jax.numpy as jnp
from jax import lax
from jax.experimental import pallas as pl
from jax.experimental.pallas import tpu as pltpu
from jax.experimental.pallas import tpu_sc as plsc

CHUNK = 128
LANES = 16
NUM_SC = 2
NUM_SUBCORES = 16
SUPER = 32


def _tc_combine(partials, x, W, self_scale, bias, gamma, beta):
    n, d_in = x.shape
    d_out = W.shape[1]

    def body(p_ref, x_ref, w_ref, s_ref, b_ref, g_ref, be_ref, o_ref):
        s = s_ref[0, 0]
        agg = p_ref[0] + p_ref[1] + x_ref[...] * s
        tmp = jnp.dot(agg, w_ref[...],
                      preferred_element_type=jnp.float32) + b_ref[...]
        mean = jnp.mean(tmp, axis=0, keepdims=True)
        var = jnp.mean((tmp - mean) * (tmp - mean), axis=0, keepdims=True)
        inv = lax.rsqrt(var + 1e-5)
        o_ref[...] = (tmp - mean) * inv * g_ref[...] + be_ref[...]

    return pl.pallas_call(
        body,
        in_specs=[
            pl.BlockSpec(memory_space=pltpu.VMEM),
            pl.BlockSpec(memory_space=pltpu.VMEM),
            pl.BlockSpec(memory_space=pltpu.VMEM),
            pl.BlockSpec(memory_space=pltpu.SMEM),
            pl.BlockSpec(memory_space=pltpu.VMEM),
            pl.BlockSpec(memory_space=pltpu.VMEM),
            pl.BlockSpec(memory_space=pltpu.VMEM),
        ],
        out_shape=jax.ShapeDtypeStruct((n, d_out), jnp.float32),
    )(partials, x, W, self_scale, bias, gamma, beta)


def _make_sc_aggregate(n_nodes, d, n_chunks_per_worker, alpha_len):
    nc, ns = NUM_SC, NUM_SUBCORES
    rows_per_tile = (n_nodes // ns) // 8 * 8
    rows_rem = n_nodes - rows_per_tile * ns
    mesh = plsc.VectorSubcoreMesh(core_axis_name="c", subcore_axis_name="s",
                                  num_cores=nc, num_subcores=ns)

    n_chunks = n_chunks_per_worker
    n_sup = n_chunks // SUPER
    assert n_chunks == n_sup * SUPER

    @functools.partial(
        pl.kernel,
        mesh=mesh,
        compiler_params=pltpu.CompilerParams(needs_layout_passes=False),
        out_type=jax.ShapeDtypeStruct((nc, n_nodes, d), jnp.float32),
        scratch_types=[
            pltpu.VMEM((SUPER, CHUNK), jnp.int32),
            pltpu.VMEM((SUPER, CHUNK), jnp.int32),
            pltpu.VMEM((SUPER, CHUNK), jnp.int32),
            pltpu.VMEM((CHUNK, d), jnp.float32),
            pltpu.VMEM((CHUNK, d), jnp.float32),
            pltpu.VMEM((alpha_len,), jnp.float32),
            pltpu.VMEM_SHARED((n_nodes, d), jnp.float32),
            pltpu.SemaphoreType.DMA,
            pltpu.SemaphoreType.DMA,
        ],
    )
    def sc_agg(x_hbm, src_hbm, dst_hbm, et_hbm, alpha_hbm, out_hbm,
               src_v, dst_v, et_v, rows0, rows1, alpha_v, acc_sh, sem,
               sem_s):
        cid = lax.axis_index("c")
        sid = lax.axis_index("s")
        wid = sid * nc + cid

        def zero_rows(i, _):
            for k in range(d // LANES):
                rows0[i, pl.ds(k * LANES, LANES)] = jnp.zeros(
                    (LANES,), jnp.float32)
            return 0
        lax.fori_loop(0, CHUNK, zero_rows, 0)

        my_base = sid * rows_per_tile
        full = rows_per_tile // CHUNK
        for j in range(full):
            pltpu.sync_copy(rows0,
                            acc_sh.at[pl.ds(my_base + j * CHUNK, CHUNK)])
        tail = rows_per_tile - full * CHUNK
        if tail:
            pltpu.sync_copy(rows0.at[pl.ds(0, tail)],
                            acc_sh.at[pl.ds(my_base + full * CHUNK, tail)])
        if rows_rem:
            @pl.when(sid == 0)
            def _():
                pltpu.sync_copy(
                    rows0.at[pl.ds(0, rows_rem)],
                    acc_sh.at[pl.ds(ns * rows_per_tile, rows_rem)])

        pltpu.sync_copy(alpha_hbm, alpha_v)
        plsc.subcore_barrier()

        def scale_and_scatter(g, buf):
            def scale_grp(grp, _):
                idx = et_v[g, pl.ds(grp * LANES, LANES)]
                a16 = plsc.load_gather(alpha_v, [idx])
                for j in range(LANES):
                    a_s = a16[j]
                    row = grp * LANES + j
                    for k in range(d // LANES):
                        sl = pl.ds(k * LANES, LANES)
                        buf[row, sl] = buf[row, sl] * a_s
                return 0
            lax.fori_loop(0, CHUNK // LANES, scale_grp, 0)
            pltpu.async_copy(buf, acc_sh.at[dst_v.at[g]], sem_s, add=True)

        def outer_s(s, _):
            base_c = wid * n_chunks + s * SUPER
            pltpu.sync_copy(src_hbm.at[pl.ds(base_c, SUPER)], src_v)
            pltpu.sync_copy(dst_hbm.at[pl.ds(base_c, SUPER)], dst_v)
            pltpu.sync_copy(et_hbm.at[pl.ds(base_c, SUPER)], et_v)
            pltpu.async_copy(x_hbm.at[src_v.at[0]], rows0, sem)

            def inner(g2, _):
                g0 = g2 * 2
                for b in range(2):
                    g = g0 + b
                    cur = rows0 if b == 0 else rows1
                    nxt = rows1 if b == 0 else rows0

                    @pl.when(g > 0)
                    def _():
                        pltpu.make_async_copy(
                            nxt, acc_sh.at[dst_v.at[g]], sem_s).wait()

                    @pl.when(g + 1 < SUPER)
                    def _():
                        pltpu.async_copy(x_hbm.at[src_v.at[g + 1]], nxt,
                                         sem)
                    pltpu.make_async_copy(x_hbm.at[src_v.at[g]], cur,
                                          sem).wait()
                    scale_and_scatter(g, cur)
                return 0
            lax.fori_loop(0, SUPER // 2, inner, 0)
            pltpu.make_async_copy(rows1, acc_sh.at[dst_v.at[SUPER - 1]],
                                  sem_s).wait()
            return 0
        lax.fori_loop(0, n_sup, outer_s, 0)

        plsc.subcore_barrier()
        pltpu.sync_copy(acc_sh.at[pl.ds(my_base, rows_per_tile)],
                        out_hbm.at[cid, pl.ds(my_base, rows_per_tile)])
        if rows_rem:
            @pl.when(sid == 0)
            def _():
                pltpu.sync_copy(
                    acc_sh.at[pl.ds(ns * rows_per_tile, rows_rem)],
                    out_hbm.at[cid, pl.ds(ns * rows_per_tile, rows_rem)])

    return sc_agg


def kernel(x, r, edge_index, edge_type, W, alpha, bias, gamma, beta):
    n, d_in = x.shape
    e = edge_index.shape[1]
    self_rel = r.shape[0] - 1

    n_workers = NUM_SC * NUM_SUBCORES

    row, col = edge_index[0], edge_index[1]
    src = jnp.concatenate([col, row])
    dst = jnp.concatenate([row, col])
    et2 = jnp.concatenate([edge_type, edge_type])

    pad_rel = alpha.shape[0]
    total = 2 * e
    chunks_per_worker = -(-total // (n_workers * CHUNK))
    chunks_per_worker = -(-chunks_per_worker // SUPER) * SUPER
    per_worker = chunks_per_worker * CHUNK
    padded = per_worker * n_workers
    pad = padded - total
    spread = jnp.arange(pad, dtype=jnp.int32) % n
    src = jnp.concatenate([src, spread]).reshape(-1, CHUNK)
    dst = jnp.concatenate([dst, spread]).reshape(-1, CHUNK)
    et2 = jnp.pad(et2, (0, pad), constant_values=pad_rel).reshape(-1, CHUNK)

    alpha_len = -(-(pad_rel + 1) // LANES) * LANES
    alpha_vec = jnp.pad(alpha[:, 0], (0, alpha_len - pad_rel))

    sc_agg = _make_sc_aggregate(n, d_in, per_worker // CHUNK, alpha_len)
    partials = sc_agg(x, src, dst, et2, alpha_vec)

    self_scale = (2.0 * alpha[self_rel, 0]).reshape(1, 1)
    out = _tc_combine(partials, x, W, self_scale, bias, gamma, beta)
    return (out, r)
